# Initial kernel scaffold; baseline (speedup 1.0000x reference)
import jax
import jax.numpy as jnp
from jax.experimental import pallas as pl


def kernel(params, x, pos, typef, v, edge_index):
    raise NotImplementedError("write your pallas kernel here")


# speedup vs baseline: 1.2382x; 1.2382x over previous
import functools

import jax
import jax.numpy as jnp
from jax.experimental import pallas as pl

_INTERPRET = False

H = 256
BE = 1024
BN = 1024


def _erf(y):
    a = jnp.abs(y)
    t = _rcp(1.0 + 0.3275911 * a)
    poly = ((((1.061405429 * t - 1.453152027) * t + 1.421413741) * t
             - 0.284496736) * t + 0.254829592) * t
    e = 1.0 - poly * jnp.exp(-a * a)
    return jnp.where(y < 0, -e, e)


def _gelu(x):
    return 0.5 * x * (1.0 + _erf(x * 0.7071067811865476))


def _rcp(x):
    r = 1.0 / x
    return r * (2.0 - x * r)


def _rsqrt(x):
    r = jax.lax.rsqrt(x)
    return r * (1.5 - 0.5 * x * r * r)


def _ln(x, g, b):
    m = jnp.mean(x, axis=-1, keepdims=True)
    xc = x - m
    v = jnp.mean(xc * xc, axis=-1, keepdims=True)
    return xc * _rsqrt(v + 1e-5) * g + b


def _mm(a, w):
    return jnp.dot(a, w, preferred_element_type=jnp.float32,
                   precision=jax.lax.Precision.HIGHEST)


def _rep(shape):
    return pl.BlockSpec(shape, lambda i: (0,) * len(shape))


def _enc_node_body(xf, w1, b1, w2, b2, g, be, out):
    h = _gelu(_mm(xf[...], w1[...]) + b1[...])
    out[...] = _ln(_mm(h, w2[...]) + b2[...], g[...], be[...])


def _enc_node(xf, p, n_blocks, fin):
    w1p = jnp.zeros((fin, H), jnp.float32).at[:p['W1'].shape[0]].set(p['W1'])
    return pl.pallas_call(
        _enc_node_body,
        grid=(n_blocks,),
        in_specs=[
            pl.BlockSpec((BN, fin), lambda i: (i, 0)),
            _rep((fin, H)), _rep((1, H)), _rep((H, H)), _rep((1, H)),
            _rep((1, H)), _rep((1, H)),
        ],
        out_specs=pl.BlockSpec((BN, H), lambda i: (i, 0)),
        out_shape=jax.ShapeDtypeStruct((n_blocks * BN, H), jnp.float32),
        interpret=_INTERPRET,
    )(xf, w1p, p['b1'], p['W2'], p['b2'], p['g'], p['be'])


def _enc_edge_body(ps, pd, w1, b1, w2, b2, g, be, o1, o2):
    rel = pd[:, 0:3] - ps[:, 0:3]
    d2 = jnp.sum(rel * rel, axis=-1, keepdims=True)
    d2 = jnp.where(d2 > 0.0, d2, 1.0)
    w = _rsqrt(d2)
    dist = d2 * w
    w2_ = w * w
    rd = rel * w
    z = jnp.zeros_like(ps[:, 0:5])
    ef1 = jnp.concatenate([rel, dist, w, w2_, w2_ * w, w2_ * w2_, rd, z], axis=-1)
    ef2 = jnp.concatenate([-rel, dist, w, w2_, w2_ * w, w2_ * w2_, -rd, z], axis=-1)
    h1 = _gelu(_mm(ef1, w1[...]) + b1[...])
    o1[...] = _ln(_mm(h1, w2[...]) + b2[...], g[...], be[...])
    h2 = _gelu(_mm(ef2, w1[...]) + b1[...])
    o2[...] = _ln(_mm(h2, w2[...]) + b2[...], g[...], be[...])


def _enc_edge(ps, pd, p, n_blocks):
    w1p = jnp.zeros((16, H), jnp.float32).at[:11].set(p['W1'])
    return pl.pallas_call(
        _enc_edge_body,
        grid=(n_blocks,),
        in_specs=[
            pl.BlockSpec((BE, 16), lambda i: (i, 0)),
            pl.BlockSpec((BE, 16), lambda i: (i, 0)),
            _rep((16, H)), _rep((1, H)), _rep((H, H)), _rep((1, H)),
            _rep((1, H)), _rep((1, H)),
        ],
        out_specs=[pl.BlockSpec((BE, H), lambda i: (i, 0)),
                   pl.BlockSpec((BE, H), lambda i: (i, 0))],
        out_shape=[jax.ShapeDtypeStruct((n_blocks * BE, H), jnp.float32)] * 2,
        interpret=_INTERPRET,
    )(ps, pd, w1p, p['b1'], p['W2'], p['b2'], p['g'], p['be'])


def _edge_layer_body(gs, gd, e1, e2, wp, wm, w1c, b1, w2m, b2m, gm, bem,
                     w1f, b1f, w2f, b2f, gf, bef,
                     f1o, f2o, e1o, e2o):
    psum = gs[...] + gd[...]
    dm = gs[...] - gd[...]
    tp = _mm(psum, wp[...])
    tm = _mm(dm, wm[...])
    h1 = _gelu(tp + tm + _mm(e1[...], w1c[...]) + b1[...])
    h2 = _gelu(tp - tm + _mm(e2[...], w1c[...]) + b1[...])
    ea1 = _ln(_mm(h1, w2m[...]) + b2m[...], gm[...], bem[...])
    ea2 = _ln(_mm(h2, w2m[...]) + b2m[...], gm[...], bem[...])
    sym = ea1 + ea2
    df = ea1 - ea2
    hs = _gelu(_mm(sym, w1f[...]) + b1f[...])
    s_ = _ln(_mm(hs, w2f[...]) + b2f[...], gf[...], bef[...])
    f1 = df + s_
    f2 = s_ - df
    f1o[...] = f1
    f2o[...] = f2
    e1o[...] = e1[...] + f1
    e2o[...] = e2[...] + f2


def _edge_layer(gs, gd, e1, e2, lp, n_blocks):
    w1 = lp['msg']['W1']
    w1a, w1b, w1c = w1[:H], w1[H:2 * H], w1[2 * H:]
    wp = (w1a + w1b) * 0.5
    wm = (w1a - w1b) * 0.5
    eb = pl.BlockSpec((BE, H), lambda i: (i, 0))
    return pl.pallas_call(
        _edge_layer_body,
        grid=(n_blocks,),
        in_specs=[eb, eb, eb, eb,
                  _rep((H, H)), _rep((H, H)), _rep((H, H)), _rep((1, H)),
                  _rep((H, H)), _rep((1, H)), _rep((1, H)), _rep((1, H)),
                  _rep((H, H)), _rep((1, H)), _rep((H, H)), _rep((1, H)),
                  _rep((1, H)), _rep((1, H))],
        out_specs=[eb, eb, eb, eb],
        out_shape=[jax.ShapeDtypeStruct((n_blocks * BE, H), jnp.float32)] * 4,
        interpret=_INTERPRET,
    )(gs, gd, e1, e2, wp, wm, w1c, lp['msg']['b1'], lp['msg']['W2'],
      lp['msg']['b2'], lp['msg']['g'], lp['msg']['be'],
      lp['flux']['W1'], lp['flux']['b1'], lp['flux']['W2'], lp['flux']['b2'],
      lp['flux']['g'], lp['flux']['be'])


def _node_layer_body(node, aggs, invc, u1a, u1b, b1, w2, b2, g, be, out):
    agg = aggs[...] * invc[:, 0:1]
    u = _gelu(_mm(node[...], u1a[...]) + _mm(agg, u1b[...]) + b1[...])
    upd = _ln(_mm(u, w2[...]) + b2[...], g[...], be[...])
    out[...] = node[...] + upd


def _node_layer(node, aggs, invc, lp, n_blocks):
    u1 = lp['upd']['W1']
    u1a, u1b = u1[:H], u1[H:]
    nb = pl.BlockSpec((BN, H), lambda i: (i, 0))
    return pl.pallas_call(
        _node_layer_body,
        grid=(n_blocks,),
        in_specs=[nb, nb, pl.BlockSpec((BN, 128), lambda i: (i, 0)),
                  _rep((H, H)), _rep((H, H)), _rep((1, H)),
                  _rep((H, H)), _rep((1, H)), _rep((1, H)), _rep((1, H))],
        out_specs=nb,
        out_shape=jax.ShapeDtypeStruct((n_blocks * BN, H), jnp.float32),
        interpret=_INTERPRET,
    )(node, aggs, invc, u1a, u1b, lp['upd']['b1'], lp['upd']['W2'],
      lp['upd']['b2'], lp['upd']['g'], lp['upd']['be'])


def _dec_body(node, w1, b1, w2, b2, out):
    h = _gelu(_mm(node[...], w1[...]) + b1[...])
    out[...] = _mm(h, w2[...]) + b2[...]


def _decoder(node, p, n_blocks):
    nout = p['W2'].shape[1]
    w2p = jnp.zeros((H, 128), jnp.float32).at[:, :nout].set(p['W2'])
    b2p = jnp.zeros((1, 128), jnp.float32).at[0, :nout].set(p['b2'].reshape(-1))
    return pl.pallas_call(
        _dec_body,
        grid=(n_blocks,),
        in_specs=[pl.BlockSpec((BN, H), lambda i: (i, 0)),
                  _rep((H, H)), _rep((1, H)), _rep((H, 128)), _rep((1, 128))],
        out_specs=pl.BlockSpec((BN, 128), lambda i: (i, 0)),
        out_shape=jax.ShapeDtypeStruct((n_blocks * BN, 128), jnp.float32),
        interpret=_INTERPRET,
    )(node, p['W1'], p['b1'], w2p, b2p)


def kernel(params, x, pos, typef, v, edge_index):
    n = x.shape[0]
    e = edge_index.shape[1]
    half = e // 2
    nb_n = -(-n // BN)
    n_p = nb_n * BN
    nb_e = -(-half // BE)
    half_p = nb_e * BE

    def r1(a):
        return a.reshape(1, -1)

    pp = jax.tree.map(lambda a: r1(a) if a.ndim == 1 else a, params)

    s = edge_index[0, :half]
    d = edge_index[1, :half]
    s_p = jnp.pad(s, (0, half_p - half))
    d_p = jnp.pad(d, (0, half_p - half))
    pad_mask = jnp.arange(half_p) >= half
    s_sc = jnp.where(pad_mask, n, s_p)
    d_sc = jnp.where(pad_mask, n, d_p)

    nf = jnp.concatenate([x, pos, typef, v], axis=-1)
    fin = 32
    nf_p = jnp.zeros((n_p, fin), jnp.float32).at[:n, :nf.shape[1]].set(nf)
    node = _enc_node(nf_p, pp['enc_node'], nb_n, fin)

    pos16 = jnp.zeros((n, 16), jnp.float32).at[:, :3].set(pos)
    ps = jnp.take(pos16, s_p, axis=0)
    pd = jnp.take(pos16, d_p, axis=0)
    e1, e2 = _enc_edge(ps, pd, pp['enc_edge'], nb_e)

    ones = jnp.ones((half,), jnp.float32)
    cnt = (jax.ops.segment_sum(ones, s, num_segments=n) +
           jax.ops.segment_sum(ones, d, num_segments=n))
    cnt = jnp.maximum(cnt, 1.0)
    invc = jnp.zeros((n_p, 128), jnp.float32).at[:n, 0].set(1.0 / cnt)

    for lp in pp['layers']:
        gs = jnp.take(node, s_p, axis=0)
        gd = jnp.take(node, d_p, axis=0)
        f1, f2, e1, e2 = _edge_layer(gs, gd, e1, e2, lp, nb_e)
        aggs = (jax.ops.segment_sum(f1, s_sc, num_segments=n + 1)[:n] +
                jax.ops.segment_sum(f2, d_sc, num_segments=n + 1)[:n])
        aggs = jnp.pad(aggs, ((0, n_p - n), (0, 0)))
        node = _node_layer(node, aggs, invc, lp, nb_n)

    out = _decoder(node, pp['dec'], nb_n)
    return out[:n, :3]


# speedup vs baseline: 1.4622x; 1.1810x over previous
import functools

import jax
import jax.numpy as jnp
from jax.experimental import pallas as pl

_INTERPRET = False

H = 256
BE = 1024
BN = 1024


def _erf(y):
    a = jnp.abs(y)
    t = _rcp(1.0 + 0.3275911 * a)
    poly = ((((1.061405429 * t - 1.453152027) * t + 1.421413741) * t
             - 0.284496736) * t + 0.254829592) * t
    e = 1.0 - poly * jnp.exp(-a * a)
    return jnp.where(y < 0, -e, e)


def _gelu(x):
    return 0.5 * x * (1.0 + _erf(x * 0.7071067811865476))


def _rcp(x):
    r = 1.0 / x
    return r * (2.0 - x * r)


def _rsqrt(x):
    r = jax.lax.rsqrt(x)
    return r * (1.5 - 0.5 * x * r * r)


def _ln(x, g, b):
    m = jnp.mean(x, axis=-1, keepdims=True)
    xc = x - m
    v = jnp.mean(xc * xc, axis=-1, keepdims=True)
    return xc * _rsqrt(v + 1e-5) * g + b


def _mm(a, w):
    ahi = a.astype(jnp.bfloat16)
    alo = (a - ahi.astype(jnp.float32)).astype(jnp.bfloat16)
    whi = w.astype(jnp.bfloat16)
    wlo = (w - whi.astype(jnp.float32)).astype(jnp.bfloat16)
    d = functools.partial(jnp.dot, preferred_element_type=jnp.float32)
    return d(ahi, whi) + (d(ahi, wlo) + d(alo, whi))


def _rep(shape):
    return pl.BlockSpec(shape, lambda i: (0,) * len(shape))


def _enc_node_body(xf, w1, b1, w2, b2, g, be, out):
    h = _gelu(_mm(xf[...], w1[...]) + b1[...])
    out[...] = _ln(_mm(h, w2[...]) + b2[...], g[...], be[...])


def _enc_node(xf, p, n_blocks, fin):
    w1p = jnp.zeros((fin, H), jnp.float32).at[:p['W1'].shape[0]].set(p['W1'])
    return pl.pallas_call(
        _enc_node_body,
        grid=(n_blocks,),
        in_specs=[
            pl.BlockSpec((BN, fin), lambda i: (i, 0)),
            _rep((fin, H)), _rep((1, H)), _rep((H, H)), _rep((1, H)),
            _rep((1, H)), _rep((1, H)),
        ],
        out_specs=pl.BlockSpec((BN, H), lambda i: (i, 0)),
        out_shape=jax.ShapeDtypeStruct((n_blocks * BN, H), jnp.float32),
        interpret=_INTERPRET,
    )(xf, w1p, p['b1'], p['W2'], p['b2'], p['g'], p['be'])


def _enc_edge_body(ps, pd, w1, b1, w2, b2, g, be, o1, o2):
    rel = pd[:, 0:3] - ps[:, 0:3]
    d2 = jnp.sum(rel * rel, axis=-1, keepdims=True)
    d2 = jnp.where(d2 > 0.0, d2, 1.0)
    w = _rsqrt(d2)
    dist = d2 * w
    w2_ = w * w
    rd = rel * w
    z = jnp.zeros_like(ps[:, 0:5])
    ef1 = jnp.concatenate([rel, dist, w, w2_, w2_ * w, w2_ * w2_, rd, z], axis=-1)
    ef2 = jnp.concatenate([-rel, dist, w, w2_, w2_ * w, w2_ * w2_, -rd, z], axis=-1)
    h1 = _gelu(_mm(ef1, w1[...]) + b1[...])
    o1[...] = _ln(_mm(h1, w2[...]) + b2[...], g[...], be[...])
    h2 = _gelu(_mm(ef2, w1[...]) + b1[...])
    o2[...] = _ln(_mm(h2, w2[...]) + b2[...], g[...], be[...])


def _enc_edge(ps, pd, p, n_blocks):
    w1p = jnp.zeros((16, H), jnp.float32).at[:11].set(p['W1'])
    return pl.pallas_call(
        _enc_edge_body,
        grid=(n_blocks,),
        in_specs=[
            pl.BlockSpec((BE, 16), lambda i: (i, 0)),
            pl.BlockSpec((BE, 16), lambda i: (i, 0)),
            _rep((16, H)), _rep((1, H)), _rep((H, H)), _rep((1, H)),
            _rep((1, H)), _rep((1, H)),
        ],
        out_specs=[pl.BlockSpec((BE, H), lambda i: (i, 0)),
                   pl.BlockSpec((BE, H), lambda i: (i, 0))],
        out_shape=[jax.ShapeDtypeStruct((n_blocks * BE, H), jnp.float32)] * 2,
        interpret=_INTERPRET,
    )(ps, pd, w1p, p['b1'], p['W2'], p['b2'], p['g'], p['be'])


def _edge_layer_body(gs, gd, e1, e2, wp, wm, w1c, b1, w2m, b2m, gm, bem,
                     w1f, b1f, w2f, b2f, gf, bef,
                     f1o, f2o, e1o, e2o):
    psum = gs[...] + gd[...]
    dm = gs[...] - gd[...]
    tp = _mm(psum, wp[...])
    tm = _mm(dm, wm[...])
    h1 = _gelu(tp + tm + _mm(e1[...], w1c[...]) + b1[...])
    h2 = _gelu(tp - tm + _mm(e2[...], w1c[...]) + b1[...])
    ea1 = _ln(_mm(h1, w2m[...]) + b2m[...], gm[...], bem[...])
    ea2 = _ln(_mm(h2, w2m[...]) + b2m[...], gm[...], bem[...])
    sym = ea1 + ea2
    df = ea1 - ea2
    hs = _gelu(_mm(sym, w1f[...]) + b1f[...])
    s_ = _ln(_mm(hs, w2f[...]) + b2f[...], gf[...], bef[...])
    f1 = df + s_
    f2 = s_ - df
    f1o[...] = f1
    f2o[...] = f2
    e1o[...] = e1[...] + f1
    e2o[...] = e2[...] + f2


def _edge_layer(gs, gd, e1, e2, lp, n_blocks):
    w1 = lp['msg']['W1']
    w1a, w1b, w1c = w1[:H], w1[H:2 * H], w1[2 * H:]
    wp = (w1a + w1b) * 0.5
    wm = (w1a - w1b) * 0.5
    eb = pl.BlockSpec((BE, H), lambda i: (i, 0))
    return pl.pallas_call(
        _edge_layer_body,
        grid=(n_blocks,),
        in_specs=[eb, eb, eb, eb,
                  _rep((H, H)), _rep((H, H)), _rep((H, H)), _rep((1, H)),
                  _rep((H, H)), _rep((1, H)), _rep((1, H)), _rep((1, H)),
                  _rep((H, H)), _rep((1, H)), _rep((H, H)), _rep((1, H)),
                  _rep((1, H)), _rep((1, H))],
        out_specs=[eb, eb, eb, eb],
        out_shape=[jax.ShapeDtypeStruct((n_blocks * BE, H), jnp.float32)] * 4,
        interpret=_INTERPRET,
    )(gs, gd, e1, e2, wp, wm, w1c, lp['msg']['b1'], lp['msg']['W2'],
      lp['msg']['b2'], lp['msg']['g'], lp['msg']['be'],
      lp['flux']['W1'], lp['flux']['b1'], lp['flux']['W2'], lp['flux']['b2'],
      lp['flux']['g'], lp['flux']['be'])


def _node_layer_body(node, aggs, invc, u1a, u1b, b1, w2, b2, g, be, out):
    agg = aggs[...] * invc[:, 0:1]
    u = _gelu(_mm(node[...], u1a[...]) + _mm(agg, u1b[...]) + b1[...])
    upd = _ln(_mm(u, w2[...]) + b2[...], g[...], be[...])
    out[...] = node[...] + upd


def _node_layer(node, aggs, invc, lp, n_blocks):
    u1 = lp['upd']['W1']
    u1a, u1b = u1[:H], u1[H:]
    nb = pl.BlockSpec((BN, H), lambda i: (i, 0))
    return pl.pallas_call(
        _node_layer_body,
        grid=(n_blocks,),
        in_specs=[nb, nb, pl.BlockSpec((BN, 128), lambda i: (i, 0)),
                  _rep((H, H)), _rep((H, H)), _rep((1, H)),
                  _rep((H, H)), _rep((1, H)), _rep((1, H)), _rep((1, H))],
        out_specs=nb,
        out_shape=jax.ShapeDtypeStruct((n_blocks * BN, H), jnp.float32),
        interpret=_INTERPRET,
    )(node, aggs, invc, u1a, u1b, lp['upd']['b1'], lp['upd']['W2'],
      lp['upd']['b2'], lp['upd']['g'], lp['upd']['be'])


def _dec_body(node, w1, b1, w2, b2, out):
    h = _gelu(_mm(node[...], w1[...]) + b1[...])
    out[...] = _mm(h, w2[...]) + b2[...]


def _decoder(node, p, n_blocks):
    nout = p['W2'].shape[1]
    w2p = jnp.zeros((H, 128), jnp.float32).at[:, :nout].set(p['W2'])
    b2p = jnp.zeros((1, 128), jnp.float32).at[0, :nout].set(p['b2'].reshape(-1))
    return pl.pallas_call(
        _dec_body,
        grid=(n_blocks,),
        in_specs=[pl.BlockSpec((BN, H), lambda i: (i, 0)),
                  _rep((H, H)), _rep((1, H)), _rep((H, 128)), _rep((1, 128))],
        out_specs=pl.BlockSpec((BN, 128), lambda i: (i, 0)),
        out_shape=jax.ShapeDtypeStruct((n_blocks * BN, 128), jnp.float32),
        interpret=_INTERPRET,
    )(node, p['W1'], p['b1'], w2p, b2p)


def kernel(params, x, pos, typef, v, edge_index):
    n = x.shape[0]
    e = edge_index.shape[1]
    half = e // 2
    nb_n = -(-n // BN)
    n_p = nb_n * BN
    nb_e = -(-half // BE)
    half_p = nb_e * BE

    def r1(a):
        return a.reshape(1, -1)

    pp = jax.tree.map(lambda a: r1(a) if a.ndim == 1 else a, params)

    s = edge_index[0, :half]
    d = edge_index[1, :half]
    s_p = jnp.pad(s, (0, half_p - half))
    d_p = jnp.pad(d, (0, half_p - half))
    pad_mask = jnp.arange(half_p) >= half
    s_sc = jnp.where(pad_mask, n, s_p)
    d_sc = jnp.where(pad_mask, n, d_p)

    nf = jnp.concatenate([x, pos, typef, v], axis=-1)
    fin = 32
    nf_p = jnp.zeros((n_p, fin), jnp.float32).at[:n, :nf.shape[1]].set(nf)
    node = _enc_node(nf_p, pp['enc_node'], nb_n, fin)

    pos16 = jnp.zeros((n, 16), jnp.float32).at[:, :3].set(pos)
    ps = jnp.take(pos16, s_p, axis=0)
    pd = jnp.take(pos16, d_p, axis=0)
    e1, e2 = _enc_edge(ps, pd, pp['enc_edge'], nb_e)

    ones = jnp.ones((half,), jnp.float32)
    cnt = (jax.ops.segment_sum(ones, s, num_segments=n) +
           jax.ops.segment_sum(ones, d, num_segments=n))
    cnt = jnp.maximum(cnt, 1.0)
    invc = jnp.zeros((n_p, 128), jnp.float32).at[:n, 0].set(1.0 / cnt)

    for lp in pp['layers']:
        gs = jnp.take(node, s_p, axis=0)
        gd = jnp.take(node, d_p, axis=0)
        f1, f2, e1, e2 = _edge_layer(gs, gd, e1, e2, lp, nb_e)
        aggs = (jax.ops.segment_sum(f1, s_sc, num_segments=n + 1)[:n] +
                jax.ops.segment_sum(f2, d_sc, num_segments=n + 1)[:n])
        aggs = jnp.pad(aggs, ((0, n_p - n), (0, 0)))
        node = _node_layer(node, aggs, invc, lp, nb_n)

    out = _decoder(node, pp['dec'], nb_n)
    return out[:n, :3]


# speedup vs baseline: 1.8854x; 1.2894x over previous
import functools

import jax
import jax.numpy as jnp
from jax import lax
from jax.experimental import pallas as pl
from jax.experimental.pallas import tpu as pltpu
from jax.experimental.pallas import tpu_sc as plsc

_INTERPRET = False

H = 256
BE = 1024
BN = 1024

_NC = 2
_NS = 16
_KC = 128


def _sc_mesh():
    return plsc.VectorSubcoreMesh(core_axis_name="c", subcore_axis_name="s")


def _sc_gather_pair(table, ia, ib, m, w):
    per_w = m // (_NC * _NS)
    nchunks = per_w // _KC

    @functools.partial(
        pl.kernel, mesh=_sc_mesh(),
        out_type=[jax.ShapeDtypeStruct((m, w), jnp.float32)] * 2,
        scratch_types=[pltpu.VMEM((_KC,), jnp.int32),
                       pltpu.VMEM((_KC,), jnp.int32),
                       pltpu.VMEM((_KC, w), jnp.float32),
                       pltpu.VMEM((_KC, w), jnp.float32),
                       pltpu.SemaphoreType.DMA,
                       pltpu.SemaphoreType.DMA],
        interpret=_INTERPRET,
    )
    def k(tab_h, ia_h, ib_h, oa_h, ob_h, iav, ibv, rav, rbv, sema, semb):
        wid = lax.axis_index("s") * _NC + lax.axis_index("c")
        base = wid * per_w

        def body(i, _):
            off = base + i * _KC
            pltpu.sync_copy(ia_h.at[pl.ds(off, _KC)], iav)
            pltpu.sync_copy(ib_h.at[pl.ds(off, _KC)], ibv)
            ca = pltpu.async_copy(tab_h.at[iav], rav, sema)
            cb = pltpu.async_copy(tab_h.at[ibv], rbv, semb)
            ca.wait()
            cb.wait()
            pltpu.sync_copy(rav, oa_h.at[pl.ds(off, _KC)])
            pltpu.sync_copy(rbv, ob_h.at[pl.ds(off, _KC)])
            return 0

        lax.fori_loop(0, nchunks, body, 0)

    return k(table, ia, ib)


def _sc_scatter_add(f1, f2, si, di, m, n_p):
    own = n_p // _NC
    ownp = own + 128
    per_t = m // _NS
    nchunks = per_t // _KC
    zrows = ownp // _NS
    wr = own // _NS
    zeros = jnp.zeros((ownp, H), jnp.float32)

    @functools.partial(
        pl.kernel, mesh=_sc_mesh(),
        out_type=jax.ShapeDtypeStruct((n_p, H), jnp.float32),
        scratch_types=[pltpu.VMEM((_KC,), jnp.int32),
                       pltpu.VMEM((_KC,), jnp.int32),
                       pltpu.VMEM((_KC, H), jnp.float32),
                       pltpu.VMEM_SHARED((ownp, H), jnp.float32)],
        interpret=_INTERPRET,
    )
    def k(f1_h, f2_h, si_h, di_h, z_h, out_h, giv, liv, rows, acc):
        cid = lax.axis_index("c")
        sid = lax.axis_index("s")
        lo = cid * own
        pltpu.sync_copy(z_h.at[pl.ds(sid * zrows, zrows)],
                        acc.at[pl.ds(sid * zrows, zrows)])
        plsc.subcore_barrier()
        for arr_h, idx_h in ((f1_h, si_h), (f2_h, di_h)):
            def body(i, _, arr_h=arr_h, idx_h=idx_h):
                off = sid * per_t + i * _KC
                pltpu.sync_copy(idx_h.at[pl.ds(off, _KC)], giv)

                def lbody(j, _):
                    v = giv[pl.ds(j * 16, 16)]
                    l = v - lo
                    ok = (l >= 0) & (l < own)
                    liv[pl.ds(j * 16, 16)] = jnp.where(ok, l, own)
                    return 0

                lax.fori_loop(0, _KC // 16, lbody, 0)
                pltpu.sync_copy(arr_h.at[pl.ds(off, _KC)], acc.at[liv],
                                add=True)
                return 0

            lax.fori_loop(0, nchunks, body, 0)
        plsc.subcore_barrier()
        pltpu.sync_copy(acc.at[pl.ds(sid * wr, wr)],
                        out_h.at[pl.ds(lo + sid * wr, wr)])

    return k(f1, f2, si, di, zeros)


def _sc_counts(si, di, m, n_p):
    own = n_p // _NC
    ownp = own + 128
    per_t = m // _NS
    nchunks = per_t // _KC
    zrows = ownp // _NS
    wr = own // _NS
    zeros = jnp.zeros((ownp, 128), jnp.float32)
    ones = jnp.ones((_KC, 128), jnp.float32)

    @functools.partial(
        pl.kernel, mesh=_sc_mesh(),
        out_type=jax.ShapeDtypeStruct((n_p, 128), jnp.float32),
        scratch_types=[pltpu.VMEM((_KC,), jnp.int32),
                       pltpu.VMEM((_KC,), jnp.int32),
                       pltpu.VMEM((_KC, 128), jnp.float32),
                       pltpu.VMEM_SHARED((ownp, 128), jnp.float32)],
        interpret=_INTERPRET,
    )
    def k(si_h, di_h, z_h, one_h, out_h, giv, liv, rows, acc):
        cid = lax.axis_index("c")
        sid = lax.axis_index("s")
        lo = cid * own
        pltpu.sync_copy(z_h.at[pl.ds(sid * zrows, zrows)],
                        acc.at[pl.ds(sid * zrows, zrows)])
        pltpu.sync_copy(one_h, rows)
        plsc.subcore_barrier()
        for idx_h in (si_h, di_h):
            def body(i, _, idx_h=idx_h):
                off = sid * per_t + i * _KC
                pltpu.sync_copy(idx_h.at[pl.ds(off, _KC)], giv)

                def lbody(j, _):
                    v = giv[pl.ds(j * 16, 16)]
                    l = v - lo
                    ok = (l >= 0) & (l < own)
                    liv[pl.ds(j * 16, 16)] = jnp.where(ok, l, own)
                    return 0

                lax.fori_loop(0, _KC // 16, lbody, 0)
                pltpu.sync_copy(one_h, acc.at[liv], add=True)
                return 0

            lax.fori_loop(0, nchunks, body, 0)
        plsc.subcore_barrier()
        pltpu.sync_copy(acc.at[pl.ds(sid * wr, wr)],
                        out_h.at[pl.ds(lo + sid * wr, wr)])

    return k(si, di, zeros, ones)


def _erf(y):
    a = jnp.abs(y)
    t = _rcp(1.0 + 0.3275911 * a)
    poly = ((((1.061405429 * t - 1.453152027) * t + 1.421413741) * t
             - 0.284496736) * t + 0.254829592) * t
    e = 1.0 - poly * jnp.exp(-a * a)
    return jnp.where(y < 0, -e, e)


def _gelu(x):
    return 0.5 * x * (1.0 + _erf(x * 0.7071067811865476))


def _rcp(x):
    r = 1.0 / x
    return r * (2.0 - x * r)


def _rsqrt(x):
    r = jax.lax.rsqrt(x)
    return r * (1.5 - 0.5 * x * r * r)


def _ln(x, g, b):
    m = jnp.mean(x, axis=-1, keepdims=True)
    xc = x - m
    v = jnp.mean(xc * xc, axis=-1, keepdims=True)
    return xc * _rsqrt(v + 1e-5) * g + b


def _mm(a, w):
    ahi = a.astype(jnp.bfloat16)
    alo = (a - ahi.astype(jnp.float32)).astype(jnp.bfloat16)
    whi = w.astype(jnp.bfloat16)
    wlo = (w - whi.astype(jnp.float32)).astype(jnp.bfloat16)
    d = functools.partial(jnp.dot, preferred_element_type=jnp.float32)
    return d(ahi, whi) + (d(ahi, wlo) + d(alo, whi))


def _rep(shape):
    return pl.BlockSpec(shape, lambda i: (0,) * len(shape))


def _enc_node_body(xf, w1, b1, w2, b2, g, be, out):
    h = _gelu(_mm(xf[...], w1[...]) + b1[...])
    out[...] = _ln(_mm(h, w2[...]) + b2[...], g[...], be[...])


def _enc_node(xf, p, n_blocks, fin):
    w1p = jnp.zeros((fin, H), jnp.float32).at[:p['W1'].shape[0]].set(p['W1'])
    return pl.pallas_call(
        _enc_node_body,
        grid=(n_blocks,),
        in_specs=[
            pl.BlockSpec((BN, fin), lambda i: (i, 0)),
            _rep((fin, H)), _rep((1, H)), _rep((H, H)), _rep((1, H)),
            _rep((1, H)), _rep((1, H)),
        ],
        out_specs=pl.BlockSpec((BN, H), lambda i: (i, 0)),
        out_shape=jax.ShapeDtypeStruct((n_blocks * BN, H), jnp.float32),
        interpret=_INTERPRET,
    )(xf, w1p, p['b1'], p['W2'], p['b2'], p['g'], p['be'])


def _enc_edge_body(ps, pd, w1, b1, w2, b2, g, be, o1, o2):
    rel = pd[:, 0:3] - ps[:, 0:3]
    d2 = jnp.sum(rel * rel, axis=-1, keepdims=True)
    d2 = jnp.where(d2 > 0.0, d2, 1.0)
    w = _rsqrt(d2)
    dist = d2 * w
    w2_ = w * w
    rd = rel * w
    z = jnp.zeros_like(ps[:, 0:5])
    ef1 = jnp.concatenate([rel, dist, w, w2_, w2_ * w, w2_ * w2_, rd, z], axis=-1)
    ef2 = jnp.concatenate([-rel, dist, w, w2_, w2_ * w, w2_ * w2_, -rd, z], axis=-1)
    h1 = _gelu(_mm(ef1, w1[...]) + b1[...])
    o1[...] = _ln(_mm(h1, w2[...]) + b2[...], g[...], be[...])
    h2 = _gelu(_mm(ef2, w1[...]) + b1[...])
    o2[...] = _ln(_mm(h2, w2[...]) + b2[...], g[...], be[...])


def _enc_edge(ps, pd, p, n_blocks):
    w1p = jnp.zeros((16, H), jnp.float32).at[:11].set(p['W1'])
    return pl.pallas_call(
        _enc_edge_body,
        grid=(n_blocks,),
        in_specs=[
            pl.BlockSpec((BE, 128), lambda i: (i, 0)),
            pl.BlockSpec((BE, 128), lambda i: (i, 0)),
            _rep((16, H)), _rep((1, H)), _rep((H, H)), _rep((1, H)),
            _rep((1, H)), _rep((1, H)),
        ],
        out_specs=[pl.BlockSpec((BE, H), lambda i: (i, 0)),
                   pl.BlockSpec((BE, H), lambda i: (i, 0))],
        out_shape=[jax.ShapeDtypeStruct((n_blocks * BE, H), jnp.float32)] * 2,
        interpret=_INTERPRET,
    )(ps, pd, w1p, p['b1'], p['W2'], p['b2'], p['g'], p['be'])


def _edge_layer_body(gs, gd, e1, e2, wp, wm, w1c, b1, w2m, b2m, gm, bem,
                     w1f, b1f, w2f, b2f, gf, bef,
                     f1o, f2o, e1o, e2o):
    psum = gs[...] + gd[...]
    dm = gs[...] - gd[...]
    tp = _mm(psum, wp[...])
    tm = _mm(dm, wm[...])
    h1 = _gelu(tp + tm + _mm(e1[...], w1c[...]) + b1[...])
    h2 = _gelu(tp - tm + _mm(e2[...], w1c[...]) + b1[...])
    ea1 = _ln(_mm(h1, w2m[...]) + b2m[...], gm[...], bem[...])
    ea2 = _ln(_mm(h2, w2m[...]) + b2m[...], gm[...], bem[...])
    sym = ea1 + ea2
    df = ea1 - ea2
    hs = _gelu(_mm(sym, w1f[...]) + b1f[...])
    s_ = _ln(_mm(hs, w2f[...]) + b2f[...], gf[...], bef[...])
    f1 = df + s_
    f2 = s_ - df
    f1o[...] = f1
    f2o[...] = f2
    e1o[...] = e1[...] + f1
    e2o[...] = e2[...] + f2


def _edge_layer(gs, gd, e1, e2, lp, n_blocks):
    w1 = lp['msg']['W1']
    w1a, w1b, w1c = w1[:H], w1[H:2 * H], w1[2 * H:]
    wp = (w1a + w1b) * 0.5
    wm = (w1a - w1b) * 0.5
    eb = pl.BlockSpec((BE, H), lambda i: (i, 0))
    return pl.pallas_call(
        _edge_layer_body,
        grid=(n_blocks,),
        in_specs=[eb, eb, eb, eb,
                  _rep((H, H)), _rep((H, H)), _rep((H, H)), _rep((1, H)),
                  _rep((H, H)), _rep((1, H)), _rep((1, H)), _rep((1, H)),
                  _rep((H, H)), _rep((1, H)), _rep((H, H)), _rep((1, H)),
                  _rep((1, H)), _rep((1, H))],
        out_specs=[eb, eb, eb, eb],
        out_shape=[jax.ShapeDtypeStruct((n_blocks * BE, H), jnp.float32)] * 4,
        interpret=_INTERPRET,
    )(gs, gd, e1, e2, wp, wm, w1c, lp['msg']['b1'], lp['msg']['W2'],
      lp['msg']['b2'], lp['msg']['g'], lp['msg']['be'],
      lp['flux']['W1'], lp['flux']['b1'], lp['flux']['W2'], lp['flux']['b2'],
      lp['flux']['g'], lp['flux']['be'])


def _node_layer_body(node, aggs, cnt, u1a, u1b, b1, w2, b2, g, be, out):
    agg = aggs[...] * _rcp(jnp.maximum(cnt[:, 0:1], 1.0))
    u = _gelu(_mm(node[...], u1a[...]) + _mm(agg, u1b[...]) + b1[...])
    upd = _ln(_mm(u, w2[...]) + b2[...], g[...], be[...])
    out[...] = node[...] + upd


def _node_layer(node, aggs, cnt, lp, n_blocks):
    u1 = lp['upd']['W1']
    u1a, u1b = u1[:H], u1[H:]
    nb = pl.BlockSpec((BN, H), lambda i: (i, 0))
    return pl.pallas_call(
        _node_layer_body,
        grid=(n_blocks,),
        in_specs=[nb, nb, pl.BlockSpec((BN, 128), lambda i: (i, 0)),
                  _rep((H, H)), _rep((H, H)), _rep((1, H)),
                  _rep((H, H)), _rep((1, H)), _rep((1, H)), _rep((1, H))],
        out_specs=nb,
        out_shape=jax.ShapeDtypeStruct((n_blocks * BN, H), jnp.float32),
        interpret=_INTERPRET,
    )(node, aggs, cnt, u1a, u1b, lp['upd']['b1'], lp['upd']['W2'],
      lp['upd']['b2'], lp['upd']['g'], lp['upd']['be'])


def _dec_body(node, w1, b1, w2, b2, out):
    h = _gelu(_mm(node[...], w1[...]) + b1[...])
    out[...] = _mm(h, w2[...]) + b2[...]


def _decoder(node, p, n_blocks):
    nout = p['W2'].shape[1]
    w2p = jnp.zeros((H, 128), jnp.float32).at[:, :nout].set(p['W2'])
    b2p = jnp.zeros((1, 128), jnp.float32).at[0, :nout].set(p['b2'].reshape(-1))
    return pl.pallas_call(
        _dec_body,
        grid=(n_blocks,),
        in_specs=[pl.BlockSpec((BN, H), lambda i: (i, 0)),
                  _rep((H, H)), _rep((1, H)), _rep((H, 128)), _rep((1, 128))],
        out_specs=pl.BlockSpec((BN, 128), lambda i: (i, 0)),
        out_shape=jax.ShapeDtypeStruct((n_blocks * BN, 128), jnp.float32),
        interpret=_INTERPRET,
    )(node, p['W1'], p['b1'], w2p, b2p)


def kernel(params, x, pos, typef, v, edge_index):
    n = x.shape[0]
    e = edge_index.shape[1]
    half = e // 2
    nb_n = -(-n // BN)
    n_p = nb_n * BN
    half_p = -(-half // (_NC * _NS * _KC)) * (_NC * _NS * _KC)
    nb_e = half_p // BE

    def r1(a):
        return a.reshape(1, -1)

    pp = jax.tree.map(lambda a: r1(a) if a.ndim == 1 else a, params)

    s = edge_index[0, :half]
    d = edge_index[1, :half]
    s_p = jnp.pad(s, (0, half_p - half))
    d_p = jnp.pad(d, (0, half_p - half))
    pad_mask = jnp.arange(half_p) >= half
    s_sc = jnp.where(pad_mask, n_p, s_p)
    d_sc = jnp.where(pad_mask, n_p, d_p)

    nf = jnp.concatenate([x, pos, typef, v], axis=-1)
    fin = 32
    nf_p = jnp.zeros((n_p, fin), jnp.float32).at[:n, :nf.shape[1]].set(nf)
    node = _enc_node(nf_p, pp['enc_node'], nb_n, fin)

    pos128 = jnp.zeros((n_p, 128), jnp.float32).at[:n, :3].set(pos)
    ps, pd = _sc_gather_pair(pos128, s_p, d_p, half_p, 128)
    e1, e2 = _enc_edge(ps, pd, pp['enc_edge'], nb_e)

    ones = jnp.ones((half,), jnp.float32)
    cnt0 = (jax.ops.segment_sum(ones, s, num_segments=n) +
            jax.ops.segment_sum(ones, d, num_segments=n))
    cnt = jnp.zeros((n_p, 128), jnp.float32).at[:n, 0].set(cnt0)

    for lp in pp['layers']:
        gs, gd = _sc_gather_pair(node, s_p, d_p, half_p, H)
        f1, f2, e1, e2 = _edge_layer(gs, gd, e1, e2, lp, nb_e)
        aggs = (jax.ops.segment_sum(f1, s_sc, num_segments=n_p + 1)[:n_p] +
                jax.ops.segment_sum(f2, d_sc, num_segments=n_p + 1)[:n_p])
        node = _node_layer(node, aggs, cnt, lp, nb_n)

    out = _decoder(node, pp['dec'], nb_n)
    return out[:n, :3]
